# baseline (speedup 1.0000x reference)
import functools

import jax
import jax.numpy as jnp
from jax import lax
from jax.experimental import pallas as pl
from jax.experimental.pallas import tpu as pltpu
from jax.experimental.pallas import tpu_sc as plsc

NC, NS = 2, 16
NW = NC * NS
OUT = 7
SN = 2
FINEST = 56.0
SIZES = (256, 128, 64, 32)
NBUF = 3


def _tc_bins_to_chmajor(x):
    k, bins, c = x.shape
    blk = 8

    def body(x_ref, o_ref):
        o_ref[...] = jnp.transpose(x_ref[...], (0, 2, 1))

    return pl.pallas_call(
        body,
        grid=(k // blk,),
        in_specs=[pl.BlockSpec((blk, bins, c), lambda g: (g, 0, 0))],
        out_specs=pl.BlockSpec((blk, c, bins), lambda g: (g, 0, 0)),
        out_shape=jax.ShapeDtypeStruct((k, c, bins), jnp.float32),
    )(x)


def _sc_roi_align(table, rois_p, sizes, bases, kpad, n_ch):
    rpw = kpad // NW
    mesh = plsc.VectorSubcoreMesh(
        core_axis_name="c", subcore_axis_name="s", num_cores=NC, num_subcores=NS
    )

    thr2 = []
    for l in (1, 2, 3):
        t = FINEST * (2.0 ** l - 1e-6)
        thr2.append(jnp.float32(t * t))

    @functools.partial(
        pl.kernel,
        mesh=mesh,
        out_type=jax.ShapeDtypeStruct((kpad, OUT * OUT, n_ch), jnp.float32),
        scratch_types=[
            pltpu.VMEM((rpw, 16), jnp.float32),
            pltpu.VMEM((OUT, 128), jnp.int32),
            pltpu.VMEM((128, 256), jnp.float32),
            pltpu.VMEM((128, 256), jnp.float32),
            pltpu.VMEM((128, 256), jnp.float32),
            pltpu.VMEM((OUT * OUT, 256), jnp.float32),
            pltpu.SemaphoreType.DMA,
            pltpu.SemaphoreType.DMA,
            pltpu.SemaphoreType.DMA,
        ],
    )
    def body(tab_hbm, rois_hbm, out_hbm, rois_v, idx_t,
             gbuf0, gbuf1, gbuf2, obuf,
             gsem0, gsem1, gsem2):
        gbufs = (gbuf0, gbuf1, gbuf2)
        gsems = (gsem0, gsem1, gsem2)
        wid = lax.axis_index("s") * NC + lax.axis_index("c")
        k0 = wid * rpw
        pltpu.sync_copy(rois_hbm.at[pl.ds(k0, rpw)], rois_v)

        iota = lax.iota(jnp.int32, 16)
        grid = (iota.astype(jnp.float32) + 0.5) * (1.0 / float(SN))

        def per_roi(kk, _):
            rv = rois_v[kk]
            bi = rv[0].astype(jnp.int32)
            x1, y1, x2, y2 = rv[1], rv[2], rv[3], rv[4]

            area = (x2 - x1 + 1.0) * (y2 - y1 + 1.0)
            lvl = ((area >= thr2[0]).astype(jnp.int32)
                   + (area >= thr2[1]).astype(jnp.int32)
                   + (area >= thr2[2]).astype(jnp.int32))
            w_i = jnp.where(lvl == 0, sizes[0],
                            jnp.where(lvl == 1, sizes[1],
                                      jnp.where(lvl == 2, sizes[2], sizes[3])))
            rbase = jnp.where(lvl == 0, bases[0],
                              jnp.where(lvl == 1, bases[1],
                                        jnp.where(lvl == 2, bases[2], bases[3])))
            rbase = rbase + bi * w_i * w_i
            w_f = w_i.astype(jnp.float32)
            scale = w_f * jnp.float32(1.0 / 1024.0)

            x1s = x1 * scale
            y1s = y1 * scale
            roi_w = jnp.maximum(x2 * scale - x1s, 1.0)
            roi_h = jnp.maximum(y2 * scale - y1s, 1.0)
            bin_w = roi_w * jnp.float32(1.0 / OUT)
            bin_h = roi_h * jnp.float32(1.0 / OUT)

            def bil(coord, lim_i, lim_f):
                valid = jnp.logical_and(coord >= -1.0, coord <= lim_f)
                c0 = jnp.maximum(coord, 0.0)
                low0 = c0.astype(jnp.int32)
                cond = low0 >= lim_i - 1
                low = jnp.where(cond, lim_i - 1, low0)
                high = jnp.where(cond, lim_i - 1, low0 + 1)
                cc = jnp.where(cond, lim_f - 1.0, c0)
                fr = cc - low.astype(jnp.float32)
                vf = jnp.where(valid, jnp.float32(1.0), jnp.float32(0.0))
                return vf, low, high, fr

            ys = y1s + grid * bin_h
            xs = x1s + grid * bin_w
            vy, yl, yh, fy = bil(ys, w_i, w_f)
            vx, xl, xh, fx = bil(xs, w_i, w_f)

            yblv = rbase + yl * w_i
            ybhv = rbase + yh * w_i
            wylv = (1.0 - fy) * vy
            wyhv = fy * vy
            xw_lo = (1.0 - fx) * vx * jnp.float32(0.25)
            xw_hi = fx * vx * jnp.float32(0.25)

            for cy in range(OUT):
                for t in range(4):
                    sy = 2 * cy + t // 2
                    yb = yblv[sy] if t % 2 == 0 else ybhv[sy]
                    idx_t[cy, pl.ds(t * 32, 16)] = yb + xl
                    idx_t[cy, pl.ds(t * 32 + 16, 16)] = yb + xh

            xwsp = [[jnp.broadcast_to(xw_lo[2 * ox], (16,)),
                     jnp.broadcast_to(xw_lo[2 * ox + 1], (16,)),
                     jnp.broadcast_to(xw_hi[2 * ox], (16,)),
                     jnp.broadcast_to(xw_hi[2 * ox + 1], (16,))]
                    for ox in range(OUT)]

            def compute_chunk(cy, gbuf):
                wy_sp = [jnp.broadcast_to(wylv[2 * cy], (16,)),
                         jnp.broadcast_to(wyhv[2 * cy], (16,)),
                         jnp.broadcast_to(wylv[2 * cy + 1], (16,)),
                         jnp.broadcast_to(wyhv[2 * cy + 1], (16,))]

                @plsc.parallel_loop(0, 16, step=1)
                def per_ci(ci):
                    sl = pl.ds(ci * 16, 16)
                    for ox in range(OUT):
                        xw = xwsp[ox]
                        acc = None
                        for t in range(4):
                            b = t * 32 + 2 * ox
                            s = (xw[0] * gbuf[b, sl]
                                 + xw[1] * gbuf[b + 1, sl]
                                 + xw[2] * gbuf[b + 16, sl]
                                 + xw[3] * gbuf[b + 17, sl])
                            term = wy_sp[t] * s
                            acc = term if acc is None else acc + term
                        obuf[cy * OUT + ox, pl.ds(ci * 16, 16)] = acc

            gh = [None] * NBUF
            for cy in range(OUT + NBUF - 1):
                if cy < OUT:
                    b = cy % NBUF
                    gh[b] = pltpu.async_copy(
                        tab_hbm.at[idx_t.at[cy]], gbufs[b], gsems[b])
                pc = cy - (NBUF - 1)
                if pc >= 0:
                    b = pc % NBUF
                    gh[b].wait()
                    compute_chunk(pc, gbufs[b])
            pltpu.sync_copy(obuf, out_hbm.at[k0 + kk])
            return 0

        lax.fori_loop(0, rpw, per_roi, 0)

    return body(table, rois_p)


def kernel(feat0, feat1, feat2, feat3, rois):
    feats = [feat0, feat1, feat2, feat3]
    n_ch = feats[0].shape[1]
    k_rois = rois.shape[0]
    kpad = ((k_rois + NW - 1) // NW) * NW

    tabs = []
    bases = []
    off = 0
    sizes = []
    for f in feats:
        b, c, h, w = f.shape
        tabs.append(jnp.transpose(f, (0, 2, 3, 1)).reshape(-1, c))
        bases.append(off)
        sizes.append(h)
        off += b * h * w
    table = jnp.concatenate(tabs, 0)

    rois_p = jnp.zeros((kpad, 16), jnp.float32).at[:k_rois, :5].set(rois)

    out = _sc_roi_align(table, rois_p, sizes, bases, kpad, n_ch)
    out = _tc_bins_to_chmajor(out)[:k_rois]
    return out.reshape(k_rois, n_ch, OUT, OUT)


# speedup vs baseline: 1.0353x; 1.0353x over previous
import functools

import jax
import jax.numpy as jnp
from jax import lax
from jax.experimental import pallas as pl
from jax.experimental.pallas import tpu as pltpu
from jax.experimental.pallas import tpu_sc as plsc

NC, NS = 2, 16
NW = NC * NS
OUT = 7
SN = 2
FINEST = 56.0
SIZES = (256, 128, 64, 32)
NBUF = 3


def _sc_roi_align(table, rois_p, sizes, bases, kpad, n_ch):
    rpw = kpad // NW
    mesh = plsc.VectorSubcoreMesh(
        core_axis_name="c", subcore_axis_name="s", num_cores=NC, num_subcores=NS
    )

    thr2 = []
    for l in (1, 2, 3):
        t = FINEST * (2.0 ** l - 1e-6)
        thr2.append(jnp.float32(t * t))

    @functools.partial(
        pl.kernel,
        mesh=mesh,
        out_type=jax.ShapeDtypeStruct((kpad, OUT * OUT, n_ch), jnp.float32),
        scratch_types=[
            pltpu.VMEM((rpw, 16), jnp.float32),
            pltpu.VMEM((OUT, 128), jnp.int32),
            pltpu.VMEM((128, 256), jnp.float32),
            pltpu.VMEM((128, 256), jnp.float32),
            pltpu.VMEM((128, 256), jnp.float32),
            pltpu.VMEM((OUT * OUT, 256), jnp.float32),
            pltpu.SemaphoreType.DMA,
            pltpu.SemaphoreType.DMA,
            pltpu.SemaphoreType.DMA,
        ],
    )
    def body(tab_hbm, rois_hbm, out_hbm, rois_v, idx_t,
             gbuf0, gbuf1, gbuf2, obuf,
             gsem0, gsem1, gsem2):
        gbufs = (gbuf0, gbuf1, gbuf2)
        gsems = (gsem0, gsem1, gsem2)
        wid = lax.axis_index("s") * NC + lax.axis_index("c")
        k0 = wid * rpw
        pltpu.sync_copy(rois_hbm.at[pl.ds(k0, rpw)], rois_v)

        iota = lax.iota(jnp.int32, 16)
        grid = (iota.astype(jnp.float32) + 0.5) * (1.0 / float(SN))

        def per_roi(kk, _):
            rv = rois_v[kk]
            bi = rv[0].astype(jnp.int32)
            x1, y1, x2, y2 = rv[1], rv[2], rv[3], rv[4]

            area = (x2 - x1 + 1.0) * (y2 - y1 + 1.0)
            lvl = ((area >= thr2[0]).astype(jnp.int32)
                   + (area >= thr2[1]).astype(jnp.int32)
                   + (area >= thr2[2]).astype(jnp.int32))
            w_i = jnp.where(lvl == 0, sizes[0],
                            jnp.where(lvl == 1, sizes[1],
                                      jnp.where(lvl == 2, sizes[2], sizes[3])))
            rbase = jnp.where(lvl == 0, bases[0],
                              jnp.where(lvl == 1, bases[1],
                                        jnp.where(lvl == 2, bases[2], bases[3])))
            rbase = rbase + bi * w_i * w_i
            w_f = w_i.astype(jnp.float32)
            scale = w_f * jnp.float32(1.0 / 1024.0)

            x1s = x1 * scale
            y1s = y1 * scale
            roi_w = jnp.maximum(x2 * scale - x1s, 1.0)
            roi_h = jnp.maximum(y2 * scale - y1s, 1.0)
            bin_w = roi_w * jnp.float32(1.0 / OUT)
            bin_h = roi_h * jnp.float32(1.0 / OUT)

            def bil(coord, lim_i, lim_f):
                valid = jnp.logical_and(coord >= -1.0, coord <= lim_f)
                c0 = jnp.maximum(coord, 0.0)
                low0 = c0.astype(jnp.int32)
                cond = low0 >= lim_i - 1
                low = jnp.where(cond, lim_i - 1, low0)
                high = jnp.where(cond, lim_i - 1, low0 + 1)
                cc = jnp.where(cond, lim_f - 1.0, c0)
                fr = cc - low.astype(jnp.float32)
                vf = jnp.where(valid, jnp.float32(1.0), jnp.float32(0.0))
                return vf, low, high, fr

            ys = y1s + grid * bin_h
            xs = x1s + grid * bin_w
            vy, yl, yh, fy = bil(ys, w_i, w_f)
            vx, xl, xh, fx = bil(xs, w_i, w_f)

            yblv = rbase + yl * w_i
            ybhv = rbase + yh * w_i
            wylv = (1.0 - fy) * vy
            wyhv = fy * vy
            xw_lo = (1.0 - fx) * vx * jnp.float32(0.25)
            xw_hi = fx * vx * jnp.float32(0.25)

            for cy in range(OUT):
                for t in range(4):
                    sy = 2 * cy + t // 2
                    yb = yblv[sy] if t % 2 == 0 else ybhv[sy]
                    idx_t[cy, pl.ds(t * 32, 16)] = yb + xl
                    idx_t[cy, pl.ds(t * 32 + 16, 16)] = yb + xh

            xwsp = [[jnp.broadcast_to(xw_lo[2 * ox], (16,)),
                     jnp.broadcast_to(xw_lo[2 * ox + 1], (16,)),
                     jnp.broadcast_to(xw_hi[2 * ox], (16,)),
                     jnp.broadcast_to(xw_hi[2 * ox + 1], (16,))]
                    for ox in range(OUT)]

            def compute_chunk(cy, gbuf):
                wy_sp = [jnp.broadcast_to(wylv[2 * cy], (16,)),
                         jnp.broadcast_to(wyhv[2 * cy], (16,)),
                         jnp.broadcast_to(wylv[2 * cy + 1], (16,)),
                         jnp.broadcast_to(wyhv[2 * cy + 1], (16,))]

                @plsc.parallel_loop(0, 16, step=1)
                def per_ci(ci):
                    sl = pl.ds(ci * 16, 16)
                    for ox in range(OUT):
                        xw = xwsp[ox]
                        acc = None
                        for t in range(4):
                            b = t * 32 + 2 * ox
                            s = (xw[0] * gbuf[b, sl]
                                 + xw[1] * gbuf[b + 1, sl]
                                 + xw[2] * gbuf[b + 16, sl]
                                 + xw[3] * gbuf[b + 17, sl])
                            term = wy_sp[t] * s
                            acc = term if acc is None else acc + term
                        obuf[cy * OUT + ox, pl.ds(ci * 16, 16)] = acc

            gh = [None] * NBUF
            for cy in range(OUT + NBUF - 1):
                if cy < OUT:
                    b = cy % NBUF
                    gh[b] = pltpu.async_copy(
                        tab_hbm.at[idx_t.at[cy]], gbufs[b], gsems[b])
                pc = cy - (NBUF - 1)
                if pc >= 0:
                    b = pc % NBUF
                    gh[b].wait()
                    compute_chunk(pc, gbufs[b])
            pltpu.sync_copy(obuf, out_hbm.at[k0 + kk])
            return 0

        lax.fori_loop(0, rpw, per_roi, 0)

    return body(table, rois_p)


def kernel(feat0, feat1, feat2, feat3, rois):
    feats = [feat0, feat1, feat2, feat3]
    n_ch = feats[0].shape[1]
    k_rois = rois.shape[0]
    kpad = ((k_rois + NW - 1) // NW) * NW

    tabs = []
    bases = []
    off = 0
    sizes = []
    for f in feats:
        b, c, h, w = f.shape
        tabs.append(jnp.transpose(f, (0, 2, 3, 1)).reshape(-1, c))
        bases.append(off)
        sizes.append(h)
        off += b * h * w
    table = jnp.concatenate(tabs, 0)

    rois_p = jnp.zeros((kpad, 16), jnp.float32).at[:k_rois, :5].set(rois)

    out = _sc_roi_align(table, rois_p, sizes, bases, kpad, n_ch)[:k_rois]
    return out.transpose(0, 2, 1).reshape(k_rois, n_ch, OUT, OUT)
